# baseline (speedup 1.0000x reference)
import jax
import jax.numpy as jnp
from jax import lax
from jax.experimental import pallas as pl
from jax.experimental.pallas import tpu as pltpu

DEPTH = 1000
B0 = 32
GRID = 2048 // B0
NBUF = 4


def _copy(i, hbm, buf, sem, slot):
    return pltpu.make_async_copy(
        buf.at[slot], hbm.at[pl.ds(i * B0, B0)], sem.at[slot])


def _body(out1_ref, out2_hbm, buf, sem):
    i = pl.program_id(0)
    slot = lax.rem(i, NBUF)

    @pl.when(i >= NBUF)
    def _wait_prev():
        _copy(i, out2_hbm, buf, sem, slot).wait()

    out1_ref[0] = jnp.full((26, DEPTH), 1.0, jnp.float32)
    buf[slot, 0] = jnp.full((26, DEPTH), 2.0, jnp.float32)

    for s in range(NBUF):
        @pl.when(slot == s)
        def _fire(s=s):
            _copy(i, out2_hbm, buf, sem, s).start()

    @pl.when(i == GRID - 1)
    def _drain():
        for s in range(NBUF):
            _copy(i, out2_hbm, buf, sem, s).wait()


def kernel(inputs):
    return pl.pallas_call(
        _body,
        grid=(GRID,),
        out_specs=(
            pl.BlockSpec((B0, 26, DEPTH), lambda i: (i, 0, 0)),
            pl.BlockSpec(memory_space=pl.ANY),
        ),
        out_shape=(
            jax.ShapeDtypeStruct((2048, 26, DEPTH), jnp.float32),
            jax.ShapeDtypeStruct((2048, 26, DEPTH), jnp.float32),
        ),
        scratch_shapes=[
            pltpu.VMEM((NBUF, B0, 26, DEPTH), jnp.float32),
            pltpu.SemaphoreType.DMA((NBUF,)),
        ],
    )()


# speedup vs baseline: 1.0948x; 1.0948x over previous
import jax
import jax.numpy as jnp
from jax import lax
from jax.experimental import pallas as pl

DEPTH = 1000
B0 = 32
GRID = 4096 // B0


def _onehot_block(idx_ref, out_ref):
    idx = idx_ref[...]
    iota = lax.broadcasted_iota(jnp.int32, out_ref.shape, 2)
    out_ref[...] = jnp.where(idx[:, :, None] == iota, 1.0, 0.0)


def kernel(inputs):
    return pl.pallas_call(
        _onehot_block,
        grid=(GRID,),
        in_specs=[pl.BlockSpec((B0, 26), lambda i: (i, 0))],
        out_specs=pl.BlockSpec((B0, 26, DEPTH), lambda i: (i, 0, 0)),
        out_shape=jax.ShapeDtypeStruct((4096, 26, DEPTH), jnp.float32),
    )(inputs)
